# baseline (speedup 1.0000x reference)
import functools

import jax
import jax.numpy as jnp
from jax import lax
from jax.experimental import pallas as pl
from jax.experimental.pallas import tpu as pltpu
from jax.experimental.pallas import tpu_sc as plsc

N = 10000
E = 320000
D = 64
NC = 2
NS = 16
NW = NC * NS
CH = 128
NCH = 80
EPT = NCH * CH
E_PAD = NW * EPT
NPAD = 10112
RPT = NPAD // NS
DUMMY = N

_mesh = plsc.VectorSubcoreMesh(core_axis_name="c", subcore_axis_name="s")
_sc_params = pltpu.CompilerParams(use_tc_tiling_on_sc=False)


@functools.partial(
    pl.kernel,
    out_type=[jax.ShapeDtypeStruct((NC, NPAD, D), jnp.float32)],
    mesh=_mesh,
    compiler_params=_sc_params,
    scratch_types=[
        pltpu.VMEM((NCH, CH), jnp.int32),
        pltpu.VMEM((NCH, CH), jnp.int32),
        pltpu.VMEM((4, CH, D), jnp.float32),
        pltpu.VMEM((CH, D), jnp.float32),
        pltpu.SemaphoreType.DMA((4,)),
        pltpu.SemaphoreType.DMA((4,)),
        pltpu.VMEM_SHARED((NPAD, D), jnp.float32),
    ],
)
def _sc_segsum(y_h, g_h, s_h, z64_h, p_h, g_v, s_v, ru, zb, sem_g, sem_s, acc):
    sid = lax.axis_index("s")
    cid = lax.axis_index("c")
    w = cid * NS + sid

    pltpu.sync_copy(g_h.at[w], g_v)
    pltpu.sync_copy(s_h.at[w], s_v)
    pltpu.sync_copy(z64_h, zb)

    base = sid * RPT
    for k in range(5):
        sz = 128 if k < 4 else RPT - 4 * 128
        pltpu.sync_copy(zb.at[pl.ds(0, sz)], acc.at[pl.ds(base + 128 * k, sz)])
    plsc.subcore_barrier()

    def gather_start(j, b):
        pltpu.async_copy(y_h.at[g_v.at[j]], ru.at[b], sem_g.at[b])

    def gather_wait(b):
        pltpu.make_async_copy(y_h.at[g_v.at[0]], ru.at[b], sem_g.at[b]).wait()

    def scatter_start(j, b):
        pltpu.async_copy(ru.at[b], acc.at[s_v.at[j]], sem_s.at[b], add=True)

    def scatter_wait(b):
        pltpu.make_async_copy(ru.at[b], acc.at[s_v.at[0]], sem_s.at[b]).wait()

    gather_start(0, 0)
    gather_start(1, 1)
    for j in range(4):
        b = j % 4
        gather_wait(b)
        scatter_start(j, b)
        if j >= 2:
            scatter_wait((j + 2) % 4)
        gather_start(j + 2, (j + 2) % 4)

    def body(i, carry):
        for b in range(4):
            j = 4 * i + b
            gather_wait(b)
            scatter_start(j, b)
            scatter_wait((b + 2) % 4)

            @pl.when(j + 2 < NCH)
            def _():
                gather_start(j + 2, (b + 2) % 4)
        return carry

    lax.fori_loop(1, NCH // 4, body, 0)
    scatter_wait(2)
    scatter_wait(3)
    plsc.subcore_barrier()

    sl = pl.ds(base, RPT)
    pltpu.sync_copy(acc.at[sl], p_h.at[cid, sl])


@functools.partial(
    pl.kernel,
    out_type=[
        jax.ShapeDtypeStruct((NC, NPAD, 16), jnp.float32),
        jax.ShapeDtypeStruct((NC, NPAD, 16), jnp.float32),
    ],
    mesh=_mesh,
    compiler_params=_sc_params,
    scratch_types=[
        pltpu.VMEM((NCH, CH), jnp.int32),
        pltpu.VMEM((NCH, CH), jnp.int32),
        pltpu.VMEM((CH, 16), jnp.float32),
        pltpu.VMEM((CH, 16), jnp.float32),
        pltpu.VMEM_SHARED((NPAD, 16), jnp.float32),
        pltpu.VMEM_SHARED((NPAD, 16), jnp.float32),
    ],
)
def _sc_counts(srcs_h, dsts_h, z16_h, o16_h, pcd_h, pcs_h,
               srcs_v, dsts_v, zc, ob, acc_cd, acc_cs):
    sid = lax.axis_index("s")
    cid = lax.axis_index("c")
    w = cid * NS + sid

    pltpu.sync_copy(srcs_h.at[w], srcs_v)
    pltpu.sync_copy(dsts_h.at[w], dsts_v)
    pltpu.sync_copy(z16_h, zc)
    pltpu.sync_copy(o16_h, ob)

    base = sid * RPT
    for k in range(5):
        sz = 128 if k < 4 else RPT - 4 * 128
        sl = pl.ds(base + 128 * k, sz)
        pltpu.sync_copy(zc.at[pl.ds(0, sz)], acc_cd.at[sl])
        pltpu.sync_copy(zc.at[pl.ds(0, sz)], acc_cs.at[sl])
    plsc.subcore_barrier()

    def body(j, carry):
        pltpu.sync_copy(ob, acc_cd.at[dsts_v.at[j]], add=True)
        pltpu.sync_copy(ob, acc_cs.at[srcs_v.at[j]], add=True)
        return carry

    lax.fori_loop(0, NCH, body, 0)
    plsc.subcore_barrier()

    sl = pl.ds(base, RPT)
    pltpu.sync_copy(acc_cd.at[sl], pcd_h.at[cid, sl])
    pltpu.sync_copy(acc_cs.at[sl], pcs_h.at[cid, sl])


def _relu(v):
    return jnp.maximum(v, 0.0)


NB = 2000
G = N // NB


def _rows(k):
    return pl.BlockSpec((NB, k), lambda i: (i, 0))


def _psum_spec(k):
    return pl.BlockSpec((NC, NB, k), lambda i: (0, i, 0))


def _full(*shape):
    nd = len(shape)
    return pl.BlockSpec(shape, lambda i: (0,) * nd)


def _tc1_body(x_ref, wu_ref, bu_ref, wd_ref, bd_ref, yu_ref, yd_ref):
    x = x_ref[...]
    yu_ref[...] = _relu(
        jnp.dot(x, wu_ref[...], preferred_element_type=jnp.float32)
        + bu_ref[...])
    yd_ref[...] = _relu(
        jnp.dot(x, wd_ref[...], preferred_element_type=jnp.float32)
        + bd_ref[...])


def _mean(p_ref, pc_ref):
    s = p_ref[0] + p_ref[1]
    c = pc_ref[0, :, 0:1] + pc_ref[1, :, 0:1]
    return s / jnp.maximum(c, 1.0)


def _tc2_body(x_ref, pu_ref, pd_ref, pcd_ref, pcs_ref, wux_ref, wua_ref,
              wdx_ref, wda_ref, w2l_ref, b2_ref, h1_ref, y2_ref):
    x = x_ref[...]
    aggu = _mean(pu_ref, pcd_ref)
    aggd = _mean(pd_ref, pcs_ref)
    xu = _relu(jnp.dot(x, wux_ref[...], preferred_element_type=jnp.float32)
               + jnp.dot(aggu, wua_ref[...], preferred_element_type=jnp.float32))
    xd = _relu(jnp.dot(x, wdx_ref[...], preferred_element_type=jnp.float32)
               + jnp.dot(aggd, wda_ref[...], preferred_element_type=jnp.float32))
    h1 = jnp.concatenate([xu, xd], axis=1)
    h1_ref[...] = h1
    y2_ref[...] = _relu(
        jnp.dot(h1, w2l_ref[...], preferred_element_type=jnp.float32)
        + b2_ref[...])


def _tc3_body(h1_ref, p2_ref, pcd_ref, w2x_ref, w2a_ref, w2lf_ref, b2_ref,
              t2_ref, y3_ref):
    agg2 = _mean(p2_ref, pcd_ref)
    t2 = _relu(
        jnp.dot(h1_ref[...], w2x_ref[...], preferred_element_type=jnp.float32)
        + jnp.dot(agg2, w2a_ref[...], preferred_element_type=jnp.float32))
    t2_ref[...] = t2
    y3_ref[...] = _relu(
        jnp.dot(t2, w2lf_ref[...], preferred_element_type=jnp.float32)
        + b2_ref[...])


def _tc4_body(t2_ref, p3_ref, pcd_ref, w2xf_ref, w2a_ref, out_ref):
    agg3 = _mean(p3_ref, pcd_ref)
    t3 = _relu(
        jnp.dot(t2_ref[...], w2xf_ref[...], preferred_element_type=jnp.float32)
        + jnp.dot(agg3, w2a_ref[...], preferred_element_type=jnp.float32))
    out_ref[:, :D] = t3
    out_ref[:, D:] = t3


def kernel(x, edge_index, W1u_lin, b1u_lin, W1u_upd, W1d_lin, b1d_lin,
           W1d_upd, W2_lin, b2_lin, W2_upd):
    f32 = jnp.float32
    src = edge_index[0].astype(jnp.int32)
    dst = edge_index[1].astype(jnp.int32)

    pad_g = jnp.zeros((E_PAD - E,), jnp.int32)
    pad_s = DUMMY + jnp.arange(E_PAD - E, dtype=jnp.int32) % (NPAD - N)
    srcg = jnp.concatenate([src, pad_g]).reshape(NW, NCH, CH)
    srcs = jnp.concatenate([src, pad_s]).reshape(NW, NCH, CH)
    dstg = jnp.concatenate([dst, pad_g]).reshape(NW, NCH, CH)
    dsts = jnp.concatenate([dst, pad_s]).reshape(NW, NCH, CH)

    z64 = jnp.zeros((CH, D), f32)
    z16 = jnp.zeros((CH, 16), f32)
    o16 = jnp.ones((CH, 16), f32)

    w1u_lin_t = W1u_lin.T
    w1d_lin_t = W1d_lin.T
    w1ux = W1u_upd[:, :128].T
    w1ua = W1u_upd[:, 128:].T
    w1dx = W1d_upd[:, :128].T
    w1da = W1d_upd[:, 128:].T
    w2l = W2_lin.T
    w2x = W2_upd[:, :128].T
    w2a = W2_upd[:, 128:].T
    w2lf = (W2_lin[:, :D] + W2_lin[:, D:]).T
    w2xf = (W2_upd[:, :D] + W2_upd[:, D:128]).T
    b1u = b1u_lin.reshape(1, D)
    b1d = b1d_lin.reshape(1, D)
    b2 = b2_lin.reshape(1, D)

    yu, yd = pl.pallas_call(
        _tc1_body,
        grid=(G,),
        in_specs=[_rows(128), _full(128, D), _full(1, D), _full(128, D),
                  _full(1, D)],
        out_specs=[_rows(D), _rows(D)],
        out_shape=[jax.ShapeDtypeStruct((N, D), f32)] * 2,
    )(x, w1u_lin_t, b1u, w1d_lin_t, b1d)

    pcd, pcs = _sc_counts(srcs, dsts, z16, o16)
    (pu,) = _sc_segsum(yu, srcg, dsts, z64)
    (pd,) = _sc_segsum(yd, dstg, srcs, z64)

    h1, y2 = pl.pallas_call(
        _tc2_body,
        grid=(G,),
        in_specs=[_rows(128), _psum_spec(D), _psum_spec(D), _psum_spec(16),
                  _psum_spec(16), _full(128, D), _full(D, D), _full(128, D),
                  _full(D, D), _full(128, D), _full(1, D)],
        out_specs=[_rows(2 * D), _rows(D)],
        out_shape=[jax.ShapeDtypeStruct((N, 2 * D), f32),
                   jax.ShapeDtypeStruct((N, D), f32)],
    )(x, pu, pd, pcd, pcs, w1ux, w1ua, w1dx, w1da, w2l, b2)

    (p2,) = _sc_segsum(y2, srcg, dsts, z64)

    t2, y3 = pl.pallas_call(
        _tc3_body,
        grid=(G,),
        in_specs=[_rows(128), _psum_spec(D), _psum_spec(16), _full(128, D),
                  _full(D, D), _full(D, D), _full(1, D)],
        out_specs=[_rows(D), _rows(D)],
        out_shape=[jax.ShapeDtypeStruct((N, D), f32),
                   jax.ShapeDtypeStruct((N, D), f32)],
    )(h1, p2, pcd, w2x, w2a, w2lf, b2)

    (p3,) = _sc_segsum(y3, srcg, dsts, z64)

    out = pl.pallas_call(
        _tc4_body,
        grid=(G,),
        in_specs=[_rows(D), _psum_spec(D), _psum_spec(16), _full(D, D),
                  _full(D, D)],
        out_specs=_rows(2 * D),
        out_shape=jax.ShapeDtypeStruct((N, 2 * D), f32),
    )(t2, p3, pcd, w2xf, w2a)
    return out


# speedup vs baseline: 2.2655x; 2.2655x over previous
import functools

import jax
import jax.numpy as jnp
from jax import lax
from jax.experimental import pallas as pl
from jax.experimental.pallas import tpu as pltpu
from jax.experimental.pallas import tpu_sc as plsc

N = 10000
E = 320000
D = 64
NC = 2
NS = 16
NW = NC * NS
CH = 128
NCH = 80
EPT = NCH * CH
E_PAD = NW * EPT
NPAD = 10016
RPT = NPAD // NS
ECH = E_PAD // (NS * CH)
DUMMY = N
HW = D // NC
RS = N // NS

_mesh = plsc.VectorSubcoreMesh(core_axis_name="c", subcore_axis_name="s")
_sc_params = pltpu.CompilerParams(use_tc_tiling_on_sc=False)


@functools.partial(
    pl.kernel,
    out_type=[jax.ShapeDtypeStruct((NC, NPAD, HW), jnp.float32)],
    mesh=_mesh,
    compiler_params=_sc_params,
    scratch_types=[
        pltpu.VMEM((ECH, CH), jnp.int32),
        pltpu.VMEM((ECH, CH), jnp.int32),
        pltpu.VMEM((4, CH, HW), jnp.float32),
        pltpu.VMEM((CH, HW), jnp.float32),
        pltpu.VMEM((RS, HW), jnp.float32),
        pltpu.SemaphoreType.DMA((4,)),
        pltpu.SemaphoreType.DMA((4,)),
        pltpu.VMEM_SHARED((N, HW), jnp.float32),
        pltpu.VMEM_SHARED((NPAD, HW), jnp.float32),
    ],
)
def _sc_segsum(y_h, g_h, s_h, z32_h, p_h, g_v, s_v, ru, zb, sbuf, sem_g,
               sem_s, tbl, acc):
    sid = lax.axis_index("s")
    cid = lax.axis_index("c")

    pltpu.sync_copy(g_h.at[sid], g_v)
    pltpu.sync_copy(s_h.at[sid], s_v)
    pltpu.sync_copy(z32_h, zb)

    pltpu.sync_copy(y_h.at[cid, pl.ds(sid * RS, RS)], sbuf)
    pltpu.sync_copy(sbuf, tbl.at[pl.ds(sid * RS, RS)])

    base = sid * RPT
    for k in range(5):
        sz = 128 if k < 4 else RPT - 4 * 128
        pltpu.sync_copy(zb.at[pl.ds(0, sz)], acc.at[pl.ds(base + 128 * k, sz)])
    plsc.subcore_barrier()

    def gather_start(j, b):
        pltpu.async_copy(tbl.at[g_v.at[j]], ru.at[b], sem_g.at[b])

    def gather_wait(b):
        pltpu.make_async_copy(tbl.at[g_v.at[0]], ru.at[b], sem_g.at[b]).wait()

    def scatter_start(j, b):
        pltpu.async_copy(ru.at[b], acc.at[s_v.at[j]], sem_s.at[b], add=True)

    def scatter_wait(b):
        pltpu.make_async_copy(ru.at[b], acc.at[s_v.at[0]], sem_s.at[b]).wait()

    gather_start(0, 0)
    gather_start(1, 1)
    for j in range(4):
        b = j % 4
        gather_wait(b)
        scatter_start(j, b)
        if j >= 2:
            scatter_wait((j + 2) % 4)
        gather_start(j + 2, (j + 2) % 4)

    def body(i, carry):
        for b in range(4):
            j = 4 * i + b
            gather_wait(b)
            scatter_start(j, b)
            scatter_wait((b + 2) % 4)

            @pl.when(j + 2 < ECH)
            def _():
                gather_start(j + 2, (b + 2) % 4)
        return carry

    lax.fori_loop(1, ECH // 4, body, 0)
    scatter_wait(2)
    scatter_wait(3)
    plsc.subcore_barrier()

    sl = pl.ds(base, RPT)
    pltpu.sync_copy(acc.at[sl], p_h.at[cid, sl])


@functools.partial(
    pl.kernel,
    out_type=[jax.ShapeDtypeStruct((NC, NPAD, 16), jnp.float32)],
    mesh=_mesh,
    compiler_params=_sc_params,
    scratch_types=[
        pltpu.VMEM((ECH, CH), jnp.int32),
        pltpu.VMEM((CH, 16), jnp.float32),
        pltpu.VMEM((CH, 16), jnp.float32),
        pltpu.VMEM_SHARED((NPAD, 16), jnp.float32),
    ],
)
def _sc_counts(s2_h, z16_h, o16_h, pc_h, s_v, zc, ob, acc_c):
    sid = lax.axis_index("s")
    cid = lax.axis_index("c")

    pltpu.sync_copy(s2_h.at[cid, sid], s_v)
    pltpu.sync_copy(z16_h, zc)
    pltpu.sync_copy(o16_h, ob)

    base = sid * RPT
    for k in range(5):
        sz = 128 if k < 4 else RPT - 4 * 128
        pltpu.sync_copy(zc.at[pl.ds(0, sz)],
                        acc_c.at[pl.ds(base + 128 * k, sz)])
    plsc.subcore_barrier()

    def body(j, carry):
        pltpu.sync_copy(ob, acc_c.at[s_v.at[j]], add=True)
        return carry

    lax.fori_loop(0, ECH, body, 0)
    plsc.subcore_barrier()

    sl = pl.ds(base, RPT)
    pltpu.sync_copy(acc_c.at[sl], pc_h.at[cid, sl])


def _relu(v):
    return jnp.maximum(v, 0.0)


NB = 2000
G = N // NB


def _rows(k):
    return pl.BlockSpec((NB, k), lambda i: (i, 0))


def _psum_spec(k):
    return pl.BlockSpec((NC, NB, k), lambda i: (0, i, 0))


def _full(*shape):
    nd = len(shape)
    return pl.BlockSpec(shape, lambda i: (0,) * nd)


def _tc1_body(x_ref, wu_ref, bu_ref, wd_ref, bd_ref, yu_ref, yd_ref):
    x = x_ref[...]
    yu_ref[...] = _relu(
        jnp.dot(x, wu_ref[...], preferred_element_type=jnp.float32)
        + bu_ref[...])
    yd_ref[...] = _relu(
        jnp.dot(x, wd_ref[...], preferred_element_type=jnp.float32)
        + bd_ref[...])


def _mean(p_ref, pc_ref, ci):
    s = jnp.concatenate([p_ref[0], p_ref[1]], axis=1)
    c = pc_ref[ci, :, 0:1]
    return s / jnp.maximum(c, 1.0)


def _tc2_body(x_ref, pu_ref, pd_ref, pc_ref, wux_ref, wua_ref,
              wdx_ref, wda_ref, w2l_ref, b2_ref, h1_ref, y2_ref):
    x = x_ref[...]
    aggu = _mean(pu_ref, pc_ref, 0)
    aggd = _mean(pd_ref, pc_ref, 1)
    xu = _relu(jnp.dot(x, wux_ref[...], preferred_element_type=jnp.float32)
               + jnp.dot(aggu, wua_ref[...], preferred_element_type=jnp.float32))
    xd = _relu(jnp.dot(x, wdx_ref[...], preferred_element_type=jnp.float32)
               + jnp.dot(aggd, wda_ref[...], preferred_element_type=jnp.float32))
    h1 = jnp.concatenate([xu, xd], axis=1)
    h1_ref[...] = h1
    y2_ref[...] = _relu(
        jnp.dot(h1, w2l_ref[...], preferred_element_type=jnp.float32)
        + b2_ref[...])


def _tc3_body(h1_ref, p2_ref, pc_ref, w2x_ref, w2a_ref, w2lf_ref, b2_ref,
              t2_ref, y3_ref):
    agg2 = _mean(p2_ref, pc_ref, 0)
    t2 = _relu(
        jnp.dot(h1_ref[...], w2x_ref[...], preferred_element_type=jnp.float32)
        + jnp.dot(agg2, w2a_ref[...], preferred_element_type=jnp.float32))
    t2_ref[...] = t2
    y3_ref[...] = _relu(
        jnp.dot(t2, w2lf_ref[...], preferred_element_type=jnp.float32)
        + b2_ref[...])


def _tc4_body(t2_ref, p3_ref, pc_ref, w2xf_ref, w2a_ref, out_ref):
    agg3 = _mean(p3_ref, pc_ref, 0)
    t3 = _relu(
        jnp.dot(t2_ref[...], w2xf_ref[...], preferred_element_type=jnp.float32)
        + jnp.dot(agg3, w2a_ref[...], preferred_element_type=jnp.float32))
    out_ref[:, :D] = t3
    out_ref[:, D:] = t3


def kernel(x, edge_index, W1u_lin, b1u_lin, W1u_upd, W1d_lin, b1d_lin,
           W1d_upd, W2_lin, b2_lin, W2_upd):
    f32 = jnp.float32
    src = edge_index[0].astype(jnp.int32)
    dst = edge_index[1].astype(jnp.int32)

    pad_g = jnp.zeros((E_PAD - E,), jnp.int32)
    pad_s = DUMMY + jnp.arange(E_PAD - E, dtype=jnp.int32) % (NPAD - N)
    srcg = jnp.concatenate([src, pad_g]).reshape(NS, ECH, CH)
    srcs = jnp.concatenate([src, pad_s]).reshape(NS, ECH, CH)
    dstg = jnp.concatenate([dst, pad_g]).reshape(NS, ECH, CH)
    dsts = jnp.concatenate([dst, pad_s]).reshape(NS, ECH, CH)
    s2 = jnp.stack([dsts, srcs])

    z32 = jnp.zeros((CH, HW), f32)
    z16 = jnp.zeros((CH, 16), f32)
    o16 = jnp.ones((CH, 16), f32)

    def halves(y):
        return jnp.stack([y[:, :HW], y[:, HW:]])

    w1u_lin_t = W1u_lin.T
    w1d_lin_t = W1d_lin.T
    w1ux = W1u_upd[:, :128].T
    w1ua = W1u_upd[:, 128:].T
    w1dx = W1d_upd[:, :128].T
    w1da = W1d_upd[:, 128:].T
    w2l = W2_lin.T
    w2x = W2_upd[:, :128].T
    w2a = W2_upd[:, 128:].T
    w2lf = (W2_lin[:, :D] + W2_lin[:, D:]).T
    w2xf = (W2_upd[:, :D] + W2_upd[:, D:128]).T
    b1u = b1u_lin.reshape(1, D)
    b1d = b1d_lin.reshape(1, D)
    b2 = b2_lin.reshape(1, D)

    yu, yd = pl.pallas_call(
        _tc1_body,
        grid=(G,),
        in_specs=[_rows(128), _full(128, D), _full(1, D), _full(128, D),
                  _full(1, D)],
        out_specs=[_rows(D), _rows(D)],
        out_shape=[jax.ShapeDtypeStruct((N, D), f32)] * 2,
    )(x, w1u_lin_t, b1u, w1d_lin_t, b1d)

    (pc,) = _sc_counts(s2, z16, o16)
    (pu,) = _sc_segsum(halves(yu), srcg, dsts, z32)
    (pd,) = _sc_segsum(halves(yd), dstg, srcs, z32)

    h1, y2 = pl.pallas_call(
        _tc2_body,
        grid=(G,),
        in_specs=[_rows(128), _psum_spec(HW), _psum_spec(HW), _psum_spec(16),
                  _full(128, D), _full(D, D), _full(128, D),
                  _full(D, D), _full(128, D), _full(1, D)],
        out_specs=[_rows(2 * D), _rows(D)],
        out_shape=[jax.ShapeDtypeStruct((N, 2 * D), f32),
                   jax.ShapeDtypeStruct((N, D), f32)],
    )(x, pu, pd, pc, w1ux, w1ua, w1dx, w1da, w2l, b2)

    (p2,) = _sc_segsum(halves(y2), srcg, dsts, z32)

    t2, y3 = pl.pallas_call(
        _tc3_body,
        grid=(G,),
        in_specs=[_rows(128), _psum_spec(HW), _psum_spec(16), _full(128, D),
                  _full(D, D), _full(D, D), _full(1, D)],
        out_specs=[_rows(D), _rows(D)],
        out_shape=[jax.ShapeDtypeStruct((N, D), f32),
                   jax.ShapeDtypeStruct((N, D), f32)],
    )(h1, p2, pc, w2x, w2a, w2lf, b2)

    (p3,) = _sc_segsum(halves(y3), srcg, dsts, z32)

    out = pl.pallas_call(
        _tc4_body,
        grid=(G,),
        in_specs=[_rows(D), _psum_spec(HW), _psum_spec(16), _full(D, D),
                  _full(D, D)],
        out_specs=_rows(2 * D),
        out_shape=jax.ShapeDtypeStruct((N, 2 * D), f32),
    )(t2, p3, pc, w2xf, w2a)
    return out


# speedup vs baseline: 2.5585x; 1.1293x over previous
import functools

import jax
import jax.numpy as jnp
from jax import lax
from jax.experimental import pallas as pl
from jax.experimental.pallas import tpu as pltpu
from jax.experimental.pallas import tpu_sc as plsc

N = 10000
E = 320000
D = 64
NC = 2
NS = 16
NW = NC * NS
CH = 128
TCH = E // CH
CB = 156
CMAX = TCH - (NS - 1) * CB
NPAD = N
RPT = NPAD // NS
HW = D // NC
RS = N // NS

_mesh = plsc.VectorSubcoreMesh(core_axis_name="c", subcore_axis_name="s")
_sc_params = pltpu.CompilerParams(use_tc_tiling_on_sc=False)


@functools.partial(
    pl.kernel,
    out_type=[jax.ShapeDtypeStruct((NC, NPAD, HW), jnp.float32)],
    mesh=_mesh,
    compiler_params=_sc_params,
    scratch_types=[
        pltpu.VMEM((CMAX, CH), jnp.int32),
        pltpu.VMEM((CMAX, CH), jnp.int32),
        pltpu.VMEM((4, CH, HW), jnp.float32),
        pltpu.VMEM((CH, HW), jnp.float32),
        pltpu.VMEM((RS, HW), jnp.float32),
        pltpu.SemaphoreType.DMA((4,)),
        pltpu.SemaphoreType.DMA((4,)),
        pltpu.VMEM_SHARED((N, HW), jnp.float32),
        pltpu.VMEM_SHARED((NPAD, HW), jnp.float32),
    ],
)
def _sc_segsum(y_h, g_h, s_h, z32_h, p_h, g_v, s_v, ru, zb, sbuf, sem_g,
               sem_s, tbl, acc):
    sid = lax.axis_index("s")
    cid = lax.axis_index("c")

    cbase = sid * CB
    nch = jnp.where(sid == NS - 1, CMAX, CB)
    pltpu.sync_copy(g_h.at[pl.ds(cbase, CMAX)], g_v)
    pltpu.sync_copy(s_h.at[pl.ds(cbase, CMAX)], s_v)
    pltpu.sync_copy(z32_h, zb)

    pltpu.sync_copy(y_h.at[cid, pl.ds(sid * RS, RS)], sbuf)
    pltpu.sync_copy(sbuf, tbl.at[pl.ds(sid * RS, RS)])

    base = sid * RPT
    for k in range(5):
        sz = 128 if k < 4 else RPT - 4 * 128
        pltpu.sync_copy(zb.at[pl.ds(0, sz)], acc.at[pl.ds(base + 128 * k, sz)])
    plsc.subcore_barrier()

    def gather_start(j, b):
        pltpu.async_copy(tbl.at[g_v.at[j]], ru.at[b], sem_g.at[b])

    def gather_wait(b):
        pltpu.make_async_copy(tbl.at[g_v.at[0]], ru.at[b], sem_g.at[b]).wait()

    def scatter_start(j, b):
        pltpu.async_copy(ru.at[b], acc.at[s_v.at[j]], sem_s.at[b], add=True)

    def scatter_wait(b):
        pltpu.make_async_copy(ru.at[b], acc.at[s_v.at[0]], sem_s.at[b]).wait()

    gather_start(0, 0)
    gather_start(1, 1)
    for j in range(4):
        b = j % 4
        gather_wait(b)
        scatter_start(j, b)
        if j >= 2:
            scatter_wait((j + 2) % 4)
        gather_start(j + 2, (j + 2) % 4)

    def body(i, carry):
        for b in range(4):
            j = 4 * i + b
            gather_wait(b)
            scatter_start(j, b)
            scatter_wait((b + 2) % 4)

            @pl.when(j + 2 < nch)
            def _():
                gather_start(j + 2, (b + 2) % 4)
        return carry

    lax.fori_loop(1, nch // 4, body, 0)
    scatter_wait(2)
    scatter_wait(3)
    plsc.subcore_barrier()

    sl = pl.ds(base, RPT)
    pltpu.sync_copy(acc.at[sl], p_h.at[cid, sl])


@functools.partial(
    pl.kernel,
    out_type=[jax.ShapeDtypeStruct((NC, NPAD, 16), jnp.float32)],
    mesh=_mesh,
    compiler_params=_sc_params,
    scratch_types=[
        pltpu.VMEM((CMAX, CH), jnp.int32),
        pltpu.VMEM((CH, 16), jnp.float32),
        pltpu.VMEM((CH, 16), jnp.float32),
        pltpu.VMEM_SHARED((NPAD, 16), jnp.float32),
    ],
)
def _sc_counts(src_h, dst_h, z16_h, o16_h, pc_h, s_v, zc, ob, acc_c):
    sid = lax.axis_index("s")
    cid = lax.axis_index("c")

    cbase = sid * CB
    nch = jnp.where(sid == NS - 1, CMAX, CB)

    @pl.when(cid == 0)
    def _():
        pltpu.sync_copy(dst_h.at[pl.ds(cbase, CMAX)], s_v)

    @pl.when(cid == 1)
    def _():
        pltpu.sync_copy(src_h.at[pl.ds(cbase, CMAX)], s_v)

    pltpu.sync_copy(z16_h, zc)
    pltpu.sync_copy(o16_h, ob)

    base = sid * RPT
    for k in range(5):
        sz = 128 if k < 4 else RPT - 4 * 128
        pltpu.sync_copy(zc.at[pl.ds(0, sz)],
                        acc_c.at[pl.ds(base + 128 * k, sz)])
    plsc.subcore_barrier()

    def body(j, carry):
        pltpu.sync_copy(ob, acc_c.at[s_v.at[j]], add=True)
        return carry

    lax.fori_loop(0, nch, body, 0)
    plsc.subcore_barrier()

    sl = pl.ds(base, RPT)
    pltpu.sync_copy(acc_c.at[sl], pc_h.at[cid, sl])


def _relu(v):
    return jnp.maximum(v, 0.0)


NB = 2000
G = N // NB


def _rows(k):
    return pl.BlockSpec((NB, k), lambda i: (i, 0))


def _psum_spec(k):
    return pl.BlockSpec((NC, NB, k), lambda i: (0, i, 0))


def _full(*shape):
    nd = len(shape)
    return pl.BlockSpec(shape, lambda i: (0,) * nd)


def _split(ref, v):
    ref[0] = v[:, :HW]
    ref[1] = v[:, HW:]


def _tc1_body(x_ref, wu_ref, bu_ref, wd_ref, bd_ref, yu_ref, yd_ref):
    x = x_ref[...]
    _split(yu_ref, _relu(
        jnp.dot(x, wu_ref[...], preferred_element_type=jnp.float32)
        + bu_ref[...]))
    _split(yd_ref, _relu(
        jnp.dot(x, wd_ref[...], preferred_element_type=jnp.float32)
        + bd_ref[...]))


def _mean(p_ref, pc_ref, ci):
    s = jnp.concatenate([p_ref[0], p_ref[1]], axis=1)
    c = pc_ref[ci, :, 0:1]
    return s / jnp.maximum(c, 1.0)


def _tc2_body(x_ref, pu_ref, pd_ref, pc_ref, wux_ref, wua_ref,
              wdx_ref, wda_ref, w2l_ref, b2_ref, h1_ref, y2_ref):
    x = x_ref[...]
    aggu = _mean(pu_ref, pc_ref, 0)
    aggd = _mean(pd_ref, pc_ref, 1)
    xu = _relu(jnp.dot(x, wux_ref[...], preferred_element_type=jnp.float32)
               + jnp.dot(aggu, wua_ref[...], preferred_element_type=jnp.float32))
    xd = _relu(jnp.dot(x, wdx_ref[...], preferred_element_type=jnp.float32)
               + jnp.dot(aggd, wda_ref[...], preferred_element_type=jnp.float32))
    h1 = jnp.concatenate([xu, xd], axis=1)
    h1_ref[...] = h1
    _split(y2_ref, _relu(
        jnp.dot(h1, w2l_ref[...], preferred_element_type=jnp.float32)
        + b2_ref[...]))


def _tc3_body(h1_ref, p2_ref, pc_ref, w2x_ref, w2a_ref, w2lf_ref, b2_ref,
              t2_ref, y3_ref):
    agg2 = _mean(p2_ref, pc_ref, 0)
    t2 = _relu(
        jnp.dot(h1_ref[...], w2x_ref[...], preferred_element_type=jnp.float32)
        + jnp.dot(agg2, w2a_ref[...], preferred_element_type=jnp.float32))
    t2_ref[...] = t2
    _split(y3_ref, _relu(
        jnp.dot(t2, w2lf_ref[...], preferred_element_type=jnp.float32)
        + b2_ref[...]))


def _tc4_body(t2_ref, p3_ref, pc_ref, w2xf_ref, w2a_ref, out_ref):
    agg3 = _mean(p3_ref, pc_ref, 0)
    t3 = _relu(
        jnp.dot(t2_ref[...], w2xf_ref[...], preferred_element_type=jnp.float32)
        + jnp.dot(agg3, w2a_ref[...], preferred_element_type=jnp.float32))
    out_ref[:, :D] = t3
    out_ref[:, D:] = t3


def kernel(x, edge_index, W1u_lin, b1u_lin, W1u_upd, W1d_lin, b1d_lin,
           W1d_upd, W2_lin, b2_lin, W2_upd):
    f32 = jnp.float32
    src = edge_index[0].astype(jnp.int32)
    dst = edge_index[1].astype(jnp.int32)

    src_r = src.reshape(TCH, CH)
    dst_r = dst.reshape(TCH, CH)

    z32 = jnp.zeros((CH, HW), f32)
    z16 = jnp.zeros((CH, 16), f32)
    o16 = jnp.ones((CH, 16), f32)

    w1u_lin_t = W1u_lin.T
    w1d_lin_t = W1d_lin.T
    w1ux = W1u_upd[:, :128].T
    w1ua = W1u_upd[:, 128:].T
    w1dx = W1d_upd[:, :128].T
    w1da = W1d_upd[:, 128:].T
    w2l = W2_lin.T
    w2x = W2_upd[:, :128].T
    w2a = W2_upd[:, 128:].T
    w2lf = (W2_lin[:, :D] + W2_lin[:, D:]).T
    w2xf = (W2_upd[:, :D] + W2_upd[:, D:128]).T
    b1u = b1u_lin.reshape(1, D)
    b1d = b1d_lin.reshape(1, D)
    b2 = b2_lin.reshape(1, D)

    yu, yd = pl.pallas_call(
        _tc1_body,
        grid=(G,),
        in_specs=[_rows(128), _full(128, D), _full(1, D), _full(128, D),
                  _full(1, D)],
        out_specs=[_psum_spec(HW), _psum_spec(HW)],
        out_shape=[jax.ShapeDtypeStruct((NC, N, HW), f32)] * 2,
    )(x, w1u_lin_t, b1u, w1d_lin_t, b1d)

    (pc,) = _sc_counts(src_r, dst_r, z16, o16)
    (pu,) = _sc_segsum(yu, src_r, dst_r, z32)
    (pd,) = _sc_segsum(yd, dst_r, src_r, z32)

    h1, y2 = pl.pallas_call(
        _tc2_body,
        grid=(G,),
        in_specs=[_rows(128), _psum_spec(HW), _psum_spec(HW), _psum_spec(16),
                  _full(128, D), _full(D, D), _full(128, D),
                  _full(D, D), _full(128, D), _full(1, D)],
        out_specs=[_rows(2 * D), _psum_spec(HW)],
        out_shape=[jax.ShapeDtypeStruct((N, 2 * D), f32),
                   jax.ShapeDtypeStruct((NC, N, HW), f32)],
    )(x, pu, pd, pc, w1ux, w1ua, w1dx, w1da, w2l, b2)

    (p2,) = _sc_segsum(y2, src_r, dst_r, z32)

    t2, y3 = pl.pallas_call(
        _tc3_body,
        grid=(G,),
        in_specs=[_rows(128), _psum_spec(HW), _psum_spec(16), _full(128, D),
                  _full(D, D), _full(D, D), _full(1, D)],
        out_specs=[_rows(D), _psum_spec(HW)],
        out_shape=[jax.ShapeDtypeStruct((N, D), f32),
                   jax.ShapeDtypeStruct((NC, N, HW), f32)],
    )(h1, p2, pc, w2x, w2a, w2lf, b2)

    (p3,) = _sc_segsum(y3, src_r, dst_r, z32)

    out = pl.pallas_call(
        _tc4_body,
        grid=(G,),
        in_specs=[_rows(D), _psum_spec(HW), _psum_spec(16), _full(D, D),
                  _full(D, D)],
        out_specs=_rows(2 * D),
        out_shape=jax.ShapeDtypeStruct((N, 2 * D), f32),
    )(t2, p3, pc, w2xf, w2a)
    return out
